# baseline (speedup 1.0000x reference)
import functools

import jax
import jax.numpy as jnp
from jax.experimental import pallas as pl

N = 10000
BM = 200


def _xw_kernel(x_ref, w_ref, out_ref):
    out_ref[...] = jnp.dot(x_ref[...], w_ref[...],
                           preferred_element_type=jnp.float32)


def _gc1_kernel(adj_ref, adjc_ref, s1_ref, w2_ref, b1_ref,
                h_g_ref, h_c_ref, s2g_ref, s2c_ref):
    s1 = s1_ref[...]
    w2 = w2_ref[...]
    b1 = b1_ref[...]
    h_g = jax.nn.relu(jnp.dot(adj_ref[...], s1,
                              preferred_element_type=jnp.float32) + b1)
    h_g_ref[...] = h_g
    s2g_ref[...] = jnp.dot(h_g, w2, preferred_element_type=jnp.float32)
    h_c = jax.nn.relu(jnp.dot(adjc_ref[...], s1,
                              preferred_element_type=jnp.float32) + b1)
    h_c_ref[...] = h_c
    s2c_ref[...] = jnp.dot(h_c, w2, preferred_element_type=jnp.float32)


def _log_softmax(z):
    z = z - jnp.max(z, axis=1, keepdims=True)
    return z - jnp.log(jnp.sum(jnp.exp(z), axis=1, keepdims=True))


def _gc2_kernel(adj_ref, adjc_ref, s2g_ref, s2c_ref, b2_ref,
                lg_ref, zg_ref, lc_ref, zc_ref):
    b2 = b2_ref[...]
    z_g = jnp.dot(adj_ref[...], s2g_ref[...],
                  preferred_element_type=jnp.float32) + b2
    zg_ref[...] = z_g
    lg_ref[...] = _log_softmax(z_g)
    z_c = jnp.dot(adjc_ref[...], s2c_ref[...],
                  preferred_element_type=jnp.float32) + b2
    zc_ref[...] = z_c
    lc_ref[...] = _log_softmax(z_c)


@functools.partial(jax.jit, static_argnames=())
def kernel(x, adj, adj_CNN, W1, b1, W2, b2):
    nfeat = x.shape[1]
    nhid = W1.shape[1]
    ncls = W2.shape[1]
    b1r = b1.reshape(1, nhid)
    b2r = b2.reshape(1, ncls)

    s1 = pl.pallas_call(
        _xw_kernel,
        out_shape=jax.ShapeDtypeStruct((N, nhid), jnp.float32),
    )(x, W1)

    grid = (N // BM,)
    blk_adj = pl.BlockSpec((BM, N), lambda i: (i, 0))
    blk_small = lambda r, c: pl.BlockSpec((r, c), lambda i: (0, 0))
    blk_out = pl.BlockSpec((BM, nhid), lambda i: (i, 0))

    h_g, h_c, s2g, s2c = pl.pallas_call(
        _gc1_kernel,
        grid=grid,
        in_specs=[blk_adj, blk_adj, blk_small(N, nhid),
                  blk_small(nhid, ncls), blk_small(1, nhid)],
        out_specs=[blk_out, blk_out,
                   pl.BlockSpec((BM, ncls), lambda i: (i, 0)),
                   pl.BlockSpec((BM, ncls), lambda i: (i, 0))],
        out_shape=[jax.ShapeDtypeStruct((N, nhid), jnp.float32),
                   jax.ShapeDtypeStruct((N, nhid), jnp.float32),
                   jax.ShapeDtypeStruct((N, ncls), jnp.float32),
                   jax.ShapeDtypeStruct((N, ncls), jnp.float32)],
    )(adj, adj_CNN, s1, W2, b1r)

    blk_out2 = pl.BlockSpec((BM, ncls), lambda i: (i, 0))
    lsm_g, z_g, lsm_c, z_c = pl.pallas_call(
        _gc2_kernel,
        grid=grid,
        in_specs=[blk_adj, blk_adj, blk_small(N, ncls),
                  blk_small(N, ncls), blk_small(1, ncls)],
        out_specs=[blk_out2, blk_out2, blk_out2, blk_out2],
        out_shape=[jax.ShapeDtypeStruct((N, ncls), jnp.float32)] * 4,
    )(adj, adj_CNN, s2g, s2c, b2r)

    return (lsm_g, z_g, lsm_c, z_c, h_g, h_c)


# speedup vs baseline: 1.0025x; 1.0025x over previous
import jax
import jax.numpy as jnp
from jax.experimental import pallas as pl
from jax.experimental.pallas import tpu as pltpu

N = 10000
BM = 200


def _dot(a, b):
    return jax.lax.dot(a, b, precision=jax.lax.Precision.DEFAULT,
                       preferred_element_type=jnp.float32)


def _log_softmax(z):
    z = z - jnp.max(z, axis=1, keepdims=True)
    return z - jnp.log(jnp.sum(jnp.exp(z), axis=1, keepdims=True))


def _gc1_kernel(x_ref, adj_ref, adjc_ref, w1_ref, w2_ref, b1_ref,
                hg_ref, hc_ref, s2g_ref, s2c_ref, s1_ref):
    @pl.when(pl.program_id(0) == 0)
    def _compute_s1():
        s1_ref[...] = _dot(x_ref[...], w1_ref[...])

    s1 = s1_ref[...]
    w2 = w2_ref[...]
    b1 = b1_ref[...]
    hg = jax.nn.relu(_dot(adj_ref[...], s1) + b1)
    hg_ref[...] = hg
    s2g_ref[...] = _dot(hg, w2)
    hc = jax.nn.relu(_dot(adjc_ref[...], s1) + b1)
    hc_ref[...] = hc
    s2c_ref[...] = _dot(hc, w2)


def _gc2_kernel(adj_ref, adjc_ref, s2g_ref, s2c_ref, b2_ref,
                lg_ref, zg_ref, lc_ref, zc_ref):
    b2 = b2_ref[...]
    zg = _dot(adj_ref[...], s2g_ref[...]) + b2
    zg_ref[...] = zg
    lg_ref[...] = _log_softmax(zg)
    zc = _dot(adjc_ref[...], s2c_ref[...]) + b2
    zc_ref[...] = zc
    lc_ref[...] = _log_softmax(zc)


def kernel(x, adj, adj_CNN, W1, b1, W2, b2):
    nfeat = x.shape[1]
    nhid = W1.shape[1]
    ncls = W2.shape[1]
    b1r = b1.reshape(1, nhid)
    b2r = b2.reshape(1, ncls)

    grid = (N // BM,)
    blk_adj = pl.BlockSpec((BM, N), lambda i: (i, 0))
    const = lambda r, c: pl.BlockSpec((r, c), lambda i: (0, 0))
    blk_out = lambda c: pl.BlockSpec((BM, c), lambda i: (i, 0))

    hg, hc, s2g, s2c = pl.pallas_call(
        _gc1_kernel,
        grid=grid,
        in_specs=[const(N, nfeat), blk_adj, blk_adj,
                  const(nfeat, nhid), const(nhid, ncls), const(1, nhid)],
        out_specs=[blk_out(nhid), blk_out(nhid),
                   blk_out(ncls), blk_out(ncls)],
        out_shape=[jax.ShapeDtypeStruct((N, nhid), jnp.float32),
                   jax.ShapeDtypeStruct((N, nhid), jnp.float32),
                   jax.ShapeDtypeStruct((N, ncls), jnp.float32),
                   jax.ShapeDtypeStruct((N, ncls), jnp.float32)],
        scratch_shapes=[pltpu.VMEM((N, nhid), jnp.float32)],
        compiler_params=pltpu.CompilerParams(
            dimension_semantics=("arbitrary",),
        ),
    )(x, adj, adj_CNN, W1, W2, b1r)

    lsm_g, z_g, lsm_c, z_c = pl.pallas_call(
        _gc2_kernel,
        grid=grid,
        in_specs=[blk_adj, blk_adj, const(N, ncls), const(N, ncls),
                  const(1, ncls)],
        out_specs=[blk_out(ncls)] * 4,
        out_shape=[jax.ShapeDtypeStruct((N, ncls), jnp.float32)] * 4,
        compiler_params=pltpu.CompilerParams(
            dimension_semantics=("arbitrary",),
        ),
    )(adj, adj_CNN, s2g, s2c, b2r)

    return (lsm_g, z_g, lsm_c, z_c, hg, hc)


# speedup vs baseline: 1.0342x; 1.0316x over previous
import jax
import jax.numpy as jnp
from jax.experimental import pallas as pl
from jax.experimental.pallas import tpu as pltpu

N = 10000
BM = 200
R = N // BM


def _dot(a, b):
    return jax.lax.dot(a, b, precision=jax.lax.Precision.DEFAULT,
                       preferred_element_type=jnp.float32)


def _log_softmax(z):
    z = z - jnp.max(z, axis=1, keepdims=True)
    return z - jnp.log(jnp.sum(jnp.exp(z), axis=1, keepdims=True))


def _gcn_kernel(x_ref, adj_ref, adjc_ref, w1_ref, w2_ref, b1_ref, b2_ref,
                out_ref, s1_ref, s2g_ref, s2c_ref):
    p = pl.program_id(0)
    i = pl.program_id(1)
    rows = pl.ds(i * BM, BM)

    @pl.when((p == 0) & (i == 0))
    def _compute_s1():
        s1_ref[...] = _dot(x_ref[...], w1_ref[...])

    @pl.when(p == 0)
    def _layer1():
        s1 = s1_ref[...]
        w2 = w2_ref[...]
        b1 = b1_ref[...]
        hg = jax.nn.relu(_dot(adj_ref[...], s1) + b1)
        hc = jax.nn.relu(_dot(adjc_ref[...], s1) + b1)
        out_ref[:, 0:16] = hg
        out_ref[:, 16:32] = hc
        out_ref[:, 32:64] = jnp.zeros((BM, 32), jnp.float32)
        s2g_ref[rows, :] = _dot(hg, w2)
        s2c_ref[rows, :] = _dot(hc, w2)

    @pl.when(p == 1)
    def _layer2():
        b2 = b2_ref[...]
        zg = _dot(adj_ref[...], s2g_ref[...]) + b2
        zc = _dot(adjc_ref[...], s2c_ref[...]) + b2
        out_ref[:, 0:16] = zg
        out_ref[:, 16:32] = _log_softmax(zg)
        out_ref[:, 32:48] = zc
        out_ref[:, 48:64] = _log_softmax(zc)


def kernel(x, adj, adj_CNN, W1, b1, W2, b2):
    nfeat = x.shape[1]
    nhid = W1.shape[1]
    ncls = W2.shape[1]
    b1r = b1.reshape(1, nhid)
    b2r = b2.reshape(1, ncls)

    grid = (2, R)
    blk_adj = pl.BlockSpec((BM, N), lambda p, i: (i, 0))
    const = lambda r, c: pl.BlockSpec((r, c), lambda p, i: (0, 0))

    packed = pl.pallas_call(
        _gcn_kernel,
        grid=grid,
        in_specs=[const(N, nfeat), blk_adj, blk_adj,
                  const(nfeat, nhid), const(nhid, ncls),
                  const(1, nhid), const(1, ncls)],
        out_specs=pl.BlockSpec((BM, 64), lambda p, i: (p * R + i, 0)),
        out_shape=jax.ShapeDtypeStruct((2 * N, 64), jnp.float32),
        scratch_shapes=[
            pltpu.VMEM((N, nhid), jnp.float32),
            pltpu.VMEM((N, ncls), jnp.float32),
            pltpu.VMEM((N, ncls), jnp.float32),
        ],
        compiler_params=pltpu.CompilerParams(
            dimension_semantics=("arbitrary", "arbitrary"),
        ),
    )(x, adj, adj_CNN, W1, W2, b1r, b2r)

    h_g = packed[:N, 0:16]
    h_c = packed[:N, 16:32]
    z_g = packed[N:, 0:16]
    lsm_g = packed[N:, 16:32]
    z_c = packed[N:, 32:48]
    lsm_c = packed[N:, 48:64]
    return (lsm_g, z_g, lsm_c, z_c, h_g, h_c)
